# baseline (device time: 137586 ns/iter reference)
import jax
import jax.numpy as jnp
from jax import lax
from jax.experimental import pallas as pl
from jax.experimental.pallas import tpu as pltpu

K = 32
BLOCK_ROWS = 128
BUCKETS = 256
M_PER_BUCKET = 2


def _topk_cols(x, k):
    cols = []
    for _ in range(k):
        m = jnp.max(x, axis=1, keepdims=True)
        cols.append(m)
        x = jnp.where(x == m, -jnp.inf, x)
    return jnp.concatenate(cols, axis=1)


def _partner_xor(x, d):
    rows, n = x.shape
    bit = (lax.broadcasted_iota(jnp.int32, (rows, n), 1) // d) % 2
    fwd = pltpu.roll(x, n - d, 1)
    bwd = pltpu.roll(x, d, 1)
    return jnp.where(bit == 0, fwd, bwd)


def _rev_lanes(x):
    d = x.shape[1] // 2
    while d >= 1:
        x = _partner_xor(x, d)
        d //= 2
    return x


def _candidates_body(x_ref, o_ref):
    rows, n = x_ref.shape
    depth = n // BUCKETS
    x3 = x_ref[:, :].reshape(rows, depth // 2, 2, BUCKETS)
    hi = jnp.max(x3, axis=2)
    lo = jnp.min(x3, axis=2)
    while hi.shape[1] > 1:
        s = hi.shape[1] // 2
        ha = hi.reshape(rows, s, 2, BUCKETS)
        la = lo.reshape(rows, s, 2, BUCKETS)
        hi_a, hi_b = ha[:, :, 0, :], ha[:, :, 1, :]
        lo_a, lo_b = la[:, :, 0, :], la[:, :, 1, :]
        hi = jnp.maximum(hi_a, hi_b)
        lo_w = jnp.where(hi_a >= hi_b, lo_a, lo_b)
        lo = jnp.maximum(jnp.minimum(hi_a, hi_b), lo_w)
    o_ref[:, :] = jnp.concatenate(
        [hi.reshape(rows, BUCKETS), lo.reshape(rows, BUCKETS)], axis=1
    )


def _bitonic_merge_desc(m):
    rows, k = m.shape
    pos = lax.broadcasted_iota(jnp.int32, (rows, k), 1)
    d = k // 2
    while d >= 1:
        p = _partner_xor(m, d)
        keep_max = ((pos // d) % 2) == 0
        m = jnp.where(keep_max, jnp.maximum(m, p), jnp.minimum(m, p))
        d //= 2
    return m


def _merge_body(c_ref, o_ref, send_ref, recv_ref, send_sem, recv_sem):
    my_x = lax.axis_index("x")
    my_y = lax.axis_index("y")
    my_z = lax.axis_index("z")
    partner = (1 - my_x, my_y, my_z)

    barrier = pltpu.get_barrier_semaphore()
    pl.semaphore_signal(
        barrier, inc=1, device_id=partner, device_id_type=pl.DeviceIdType.MESH
    )

    send_ref[:, :] = _topk_cols(c_ref[:, :], K)

    pl.semaphore_wait(barrier, 1)

    rdma = pltpu.make_async_remote_copy(
        src_ref=send_ref,
        dst_ref=recv_ref,
        send_sem=send_sem,
        recv_sem=recv_sem,
        device_id=partner,
        device_id_type=pl.DeviceIdType.MESH,
    )
    rdma.start()
    rdma.wait()

    m = jnp.maximum(send_ref[:, :], _rev_lanes(recv_ref[:, :]))
    o_ref[:, :] = _bitonic_merge_desc(m)


def kernel(x):
    n_rows, n_local = x.shape
    x = x.astype(jnp.float32)

    cand = pl.pallas_call(
        _candidates_body,
        grid=(n_rows // BLOCK_ROWS,),
        in_specs=[pl.BlockSpec((BLOCK_ROWS, n_local), lambda i: (i, 0))],
        out_specs=pl.BlockSpec((BLOCK_ROWS, M_PER_BUCKET * BUCKETS), lambda i: (i, 0)),
        out_shape=jax.ShapeDtypeStruct((n_rows, M_PER_BUCKET * BUCKETS), jnp.float32),
    )(x)

    return pl.pallas_call(
        _merge_body,
        out_shape=jax.ShapeDtypeStruct((n_rows, K), jnp.float32),
        in_specs=[pl.BlockSpec(memory_space=pltpu.VMEM)],
        out_specs=pl.BlockSpec(memory_space=pltpu.VMEM),
        scratch_shapes=[
            pltpu.VMEM((n_rows, K), jnp.float32),
            pltpu.VMEM((n_rows, K), jnp.float32),
            pltpu.SemaphoreType.DMA,
            pltpu.SemaphoreType.DMA,
        ],
        compiler_params=pltpu.CompilerParams(collective_id=0),
    )(cand)


# device time: 33852 ns/iter; 4.0643x vs baseline; 4.0643x over previous
import jax
import jax.numpy as jnp
from jax import lax
from jax.experimental import pallas as pl
from jax.experimental.pallas import tpu as pltpu

K = 32
BLOCK_ROWS = 128
BUCKETS = 128
M_PER_BUCKET = 2


def _topk_cols(x, k):
    cols = []
    for _ in range(k):
        m = jnp.max(x, axis=1, keepdims=True)
        cols.append(m)
        x = jnp.where(x == m, -jnp.inf, x)
    return jnp.concatenate(cols, axis=1)


def _partner_xor(x, d):
    rows, n = x.shape
    bit = (lax.broadcasted_iota(jnp.int32, (rows, n), 1) // d) % 2
    fwd = pltpu.roll(x, n - d, 1)
    bwd = pltpu.roll(x, d, 1)
    return jnp.where(bit == 0, fwd, bwd)


def _rev_lanes(x):
    d = x.shape[1] // 2
    while d >= 1:
        x = _partner_xor(x, d)
        d //= 2
    return x


def _candidates_body(x_ref, o_ref):
    rows, n = x_ref.shape
    his, los = [], []
    for k in range(0, n // 128, 2):
        a = x_ref[:, k * 128 : (k + 1) * 128]
        b = x_ref[:, (k + 1) * 128 : (k + 2) * 128]
        his.append(jnp.maximum(a, b))
        los.append(jnp.minimum(a, b))
    while len(his) > 1:
        nh, nl = [], []
        for i in range(0, len(his), 2):
            h1, l1, h2, l2 = his[i], los[i], his[i + 1], los[i + 1]
            nh.append(jnp.maximum(h1, h2))
            nl.append(
                jnp.maximum(jnp.minimum(h1, h2), jnp.where(h1 >= h2, l1, l2))
            )
        his, los = nh, nl
    o_ref[:, :128] = his[0]
    o_ref[:, 128:] = los[0]


def _bitonic_merge_desc(m):
    rows, k = m.shape
    pos = lax.broadcasted_iota(jnp.int32, (rows, k), 1)
    d = k // 2
    while d >= 1:
        p = _partner_xor(m, d)
        keep_max = ((pos // d) % 2) == 0
        m = jnp.where(keep_max, jnp.maximum(m, p), jnp.minimum(m, p))
        d //= 2
    return m


def _merge_body(c_ref, o_ref, send_ref, recv_ref, send_sem, recv_sem):
    my_x = lax.axis_index("x")
    my_y = lax.axis_index("y")
    my_z = lax.axis_index("z")
    partner = (1 - my_x, my_y, my_z)

    barrier = pltpu.get_barrier_semaphore()
    pl.semaphore_signal(
        barrier, inc=1, device_id=partner, device_id_type=pl.DeviceIdType.MESH
    )

    head = c_ref[:, :128]
    nxt = c_ref[:, 128:]
    cols = []
    for _ in range(K):
        m = jnp.max(head, axis=1, keepdims=True)
        cols.append(m)
        hit = head == m
        head = jnp.where(hit, nxt, head)
        nxt = jnp.where(hit, -jnp.inf, nxt)
    send_ref[:, :] = jnp.concatenate(cols, axis=1)

    pl.semaphore_wait(barrier, 1)

    rdma = pltpu.make_async_remote_copy(
        src_ref=send_ref,
        dst_ref=recv_ref,
        send_sem=send_sem,
        recv_sem=recv_sem,
        device_id=partner,
        device_id_type=pl.DeviceIdType.MESH,
    )
    rdma.start()
    rdma.wait()

    m = jnp.maximum(send_ref[:, :], _rev_lanes(recv_ref[:, :]))
    o_ref[:, :] = _bitonic_merge_desc(m)


def kernel(x):
    n_rows, n_local = x.shape
    x = x.astype(jnp.float32)

    cand = pl.pallas_call(
        _candidates_body,
        grid=(n_rows // BLOCK_ROWS,),
        in_specs=[pl.BlockSpec((BLOCK_ROWS, n_local), lambda i: (i, 0))],
        out_specs=pl.BlockSpec((BLOCK_ROWS, M_PER_BUCKET * BUCKETS), lambda i: (i, 0)),
        out_shape=jax.ShapeDtypeStruct((n_rows, M_PER_BUCKET * BUCKETS), jnp.float32),
    )(x)

    return pl.pallas_call(
        _merge_body,
        out_shape=jax.ShapeDtypeStruct((n_rows, K), jnp.float32),
        in_specs=[pl.BlockSpec(memory_space=pltpu.VMEM)],
        out_specs=pl.BlockSpec(memory_space=pltpu.VMEM),
        scratch_shapes=[
            pltpu.VMEM((n_rows, K), jnp.float32),
            pltpu.VMEM((n_rows, K), jnp.float32),
            pltpu.SemaphoreType.DMA,
            pltpu.SemaphoreType.DMA,
        ],
        compiler_params=pltpu.CompilerParams(collective_id=0),
    )(cand)


# device time: 25488 ns/iter; 5.3981x vs baseline; 1.3282x over previous
import jax
import jax.numpy as jnp
from jax import lax
from jax.experimental import pallas as pl
from jax.experimental.pallas import tpu as pltpu

K = 32
BLOCK_ROWS = 256
N_BLOCKS = 4
BUCKETS = 128
N_SLOTS = 2
N_STREAMS = 2


def _partner_xor(x, d):
    rows, n = x.shape
    bit = (lax.broadcasted_iota(jnp.int32, (rows, n), 1) // d) % 2
    fwd = pltpu.roll(x, n - d, 1)
    bwd = pltpu.roll(x, d, 1)
    return jnp.where(bit == 0, fwd, bwd)


def _rev_lanes(x):
    d = x.shape[1] // 2
    while d >= 1:
        x = _partner_xor(x, d)
        d //= 2
    return x


def _bitonic_merge_desc(m):
    rows, k = m.shape
    pos = lax.broadcasted_iota(jnp.int32, (rows, k), 1)
    d = k // 2
    while d >= 1:
        p = _partner_xor(m, d)
        keep_max = ((pos // d) % 2) == 0
        m = jnp.where(keep_max, jnp.maximum(m, p), jnp.minimum(m, p))
        d //= 2
    return m


def _slab_tree(xb):
    rows, n = xb.shape
    his, los = [], []
    for k in range(0, n // 128, 2):
        a = xb[:, k * 128 : (k + 1) * 128]
        b = xb[:, (k + 1) * 128 : (k + 2) * 128]
        his.append(jnp.maximum(a, b))
        los.append(jnp.minimum(a, b))
    while len(his) > 1:
        nh, nl = [], []
        for i in range(0, len(his), 2):
            h1, l1, h2, l2 = his[i], los[i], his[i + 1], los[i + 1]
            nh.append(jnp.maximum(h1, h2))
            nl.append(jnp.maximum(jnp.minimum(h1, h2), jnp.maximum(l1, l2)))
        his, los = nh, nl
    return his[0], los[0]


def _extract_half(cand_ref, send_ref, slot, half_rows):
    chunk = half_rows // N_STREAMS
    heads, nxts = [], []
    for s in range(N_STREAMS):
        r0 = slot * half_rows + s * chunk
        heads.append(cand_ref[r0 : r0 + chunk, :BUCKETS])
        nxts.append(cand_ref[r0 : r0 + chunk, BUCKETS:])
    for i in range(K):
        for s in range(N_STREAMS):
            m = jnp.max(heads[s], axis=1, keepdims=True)
            send_ref[slot, s * chunk : (s + 1) * chunk, i : i + 1] = m
            hit = heads[s] == m
            heads[s] = jnp.where(hit, nxts[s], heads[s])
            nxts[s] = jnp.where(hit, -jnp.inf, nxts[s])


def _body(x_ref, o_ref, cand_ref, send_ref, recv_ref, send_sems, recv_sems):
    my_x = lax.axis_index("x")
    my_y = lax.axis_index("y")
    my_z = lax.axis_index("z")
    partner = (1 - my_x, my_y, my_z)
    step = pl.program_id(0)
    n_rows = N_BLOCKS * BLOCK_ROWS
    half = n_rows // N_SLOTS
    barrier = pltpu.get_barrier_semaphore()

    def make_rdma(slot):
        return pltpu.make_async_remote_copy(
            src_ref=send_ref.at[slot],
            dst_ref=recv_ref.at[slot],
            send_sem=send_sems.at[slot],
            recv_sem=recv_sems.at[slot],
            device_id=partner,
            device_id_type=pl.DeviceIdType.MESH,
        )

    @pl.when(step == 0)
    def _():
        pl.semaphore_signal(
            barrier,
            inc=1,
            device_id=partner,
            device_id_type=pl.DeviceIdType.MESH,
        )

    @pl.when(step < N_BLOCKS)
    def _():
        hi, lo = _slab_tree(x_ref[:, :])
        cand_ref[pl.ds(step * BLOCK_ROWS, BLOCK_ROWS), :BUCKETS] = hi
        cand_ref[pl.ds(step * BLOCK_ROWS, BLOCK_ROWS), BUCKETS:] = lo

    @pl.when(step == N_BLOCKS // 2)
    def _():
        _extract_half(cand_ref, send_ref, 0, half)
        pl.semaphore_wait(barrier, 1)
        make_rdma(0).start()

    @pl.when(step == N_BLOCKS - 1)
    def _():
        _extract_half(cand_ref, send_ref, 1, half)
        make_rdma(1).start()
        for slot in range(N_SLOTS):
            rdma = make_rdma(slot)
            rdma.wait_send()
            rdma.wait_recv()
            m = jnp.maximum(
                send_ref[slot, :, :], _rev_lanes(recv_ref[slot, :, :])
            )
            o_ref[slot * half : (slot + 1) * half, :] = _bitonic_merge_desc(m)


def kernel(x):
    n_rows, n_local = x.shape
    x = x.astype(jnp.float32)

    return pl.pallas_call(
        _body,
        grid=(N_BLOCKS,),
        in_specs=[pl.BlockSpec((BLOCK_ROWS, n_local), lambda i: (i, 0))],
        out_specs=pl.BlockSpec((n_rows, K), lambda i: (0, 0)),
        out_shape=jax.ShapeDtypeStruct((n_rows, K), jnp.float32),
        scratch_shapes=[
            pltpu.VMEM((n_rows, 2 * BUCKETS), jnp.float32),
            pltpu.VMEM((N_SLOTS, n_rows // N_SLOTS, K), jnp.float32),
            pltpu.VMEM((N_SLOTS, n_rows // N_SLOTS, K), jnp.float32),
            pltpu.SemaphoreType.DMA((N_SLOTS,)),
            pltpu.SemaphoreType.DMA((N_SLOTS,)),
        ],
        compiler_params=pltpu.CompilerParams(collective_id=0),
    )(x)
